# baseline (speedup 1.0000x reference)
import functools

import jax
import jax.numpy as jnp
from jax import lax
from jax.experimental import pallas as pl
from jax.experimental.pallas import tpu as pltpu
from jax.experimental.pallas import tpu_sc as plsc

_TN = 1024
_NC, _NS, _L = 2, 16, 16
_NW = _NC * _NS


def _logits_kernel(x_ref, w_ref, b_ref, out_ref):
    x = x_ref[...]
    w = w_ref[...]
    b = b_ref[...]
    out_ref[...] = jax.lax.dot_general(
        w, x, (((1,), (1,)), ((), ())),
        preferred_element_type=jnp.float32) + b


def _tc_logits(x, W, b):
    n_tokens, d_model = x.shape
    n_experts = W.shape[0]
    return pl.pallas_call(
        _logits_kernel,
        grid=(n_tokens // _TN,),
        in_specs=[
            pl.BlockSpec((_TN, d_model), lambda i: (i, 0)),
            pl.BlockSpec((n_experts, d_model), lambda i: (0, 0)),
            pl.BlockSpec((n_experts, 1), lambda i: (0, 0)),
        ],
        out_specs=pl.BlockSpec((n_experts, _TN), lambda i: (0, i)),
        out_shape=jax.ShapeDtypeStruct((n_experts, n_tokens), jnp.float32),
    )(x, W, b.reshape(n_experts, 1))


def _sc_router(logits_hbm, vals_hbm, idx_hbm, lg, valv, idxv, sem):
    n_tokens = logits_hbm.shape[1]
    tpt = n_tokens // _NW
    wid = lax.axis_index("s") * _NC + lax.axis_index("c")
    base = wid * tpt

    copies = [
        pltpu.async_copy(logits_hbm.at[e, pl.ds(base, tpt)], lg.at[e], sem)
        for e in range(16)
    ]
    for c in copies:
        c.wait()

    neg_inf = jnp.full((16,), -jnp.inf, jnp.float32)
    for g in range(tpt // _L):
        ls = [lg[e, pl.ds(g * _L, _L)] for e in range(16)]
        m1 = functools.reduce(jnp.maximum, ls)
        i1 = jnp.zeros((16,), jnp.int32)
        for e in range(15, -1, -1):
            i1 = jnp.where(ls[e] == m1, e, i1)
        l2 = [jnp.where(i1 == e, neg_inf, ls[e]) for e in range(16)]
        m2 = functools.reduce(jnp.maximum, l2)
        i2 = jnp.zeros((16,), jnp.int32)
        for e in range(15, -1, -1):
            i2 = jnp.where(l2[e] == m2, e, i2)
        s = None
        for e in range(16):
            t = jnp.exp(ls[e] - m1)
            s = t if s is None else s + t
        sl = pl.ds(g * _L, _L)
        valv[0, sl] = 1.0 / s
        valv[1, sl] = jnp.exp(m2 - m1) / s
        idxv[0, sl] = i1
        idxv[1, sl] = i2

    tok = pl.ds(base, tpt)
    pltpu.sync_copy(valv.at[0], vals_hbm.at[0, tok])
    pltpu.sync_copy(valv.at[1], vals_hbm.at[1, tok])
    pltpu.sync_copy(idxv.at[0], idx_hbm.at[0, tok])
    pltpu.sync_copy(idxv.at[1], idx_hbm.at[1, tok])


def _sc_route(logits_t):
    n_experts, n_tokens = logits_t.shape
    tpt = n_tokens // _NW
    mesh = plsc.VectorSubcoreMesh(core_axis_name="c", subcore_axis_name="s")
    return pl.kernel(
        _sc_router,
        out_type=[
            jax.ShapeDtypeStruct((2, n_tokens), jnp.float32),
            jax.ShapeDtypeStruct((2, n_tokens), jnp.int32),
        ],
        mesh=mesh,
        scratch_types=[
            pltpu.VMEM((n_experts, tpt), jnp.float32),
            pltpu.VMEM((2, tpt), jnp.float32),
            pltpu.VMEM((2, tpt), jnp.int32),
            pltpu.SemaphoreType.DMA,
        ],
    )(logits_t)


def kernel(x, W, b):
    logits_t = _tc_logits(x, W, b)
    vals_t, idx_t = _sc_route(logits_t)
    return (vals_t.T, idx_t.T)
